# baseline (speedup 1.0000x reference)
import jax
import jax.numpy as jnp
from jax.experimental import pallas as pl
from jax.experimental.pallas import tpu as pltpu

N_BINS = 15
ROWS_PER_BLOCK = 1000


def _partials_kernel(lo_ref, hi_ref, probs_ref, labels_ref, out_ref):
    x = probs_ref[...]
    conf = jnp.max(x, axis=1, keepdims=True)
    col = jax.lax.broadcasted_iota(jnp.int32, x.shape, 1)
    pred = jnp.min(jnp.where(x == conf, col, x.shape[1]), axis=1, keepdims=True)
    acc = (pred == labels_ref[...]).astype(jnp.float32)

    lo = lo_ref[...]
    hi = hi_ref[...]
    onehot = ((conf > lo) & (conf <= hi)).astype(jnp.float32)
    out_ref[0, 0:1, :] = jnp.sum(onehot, axis=0, keepdims=True)
    out_ref[0, 1:2, :] = jnp.sum(onehot * acc, axis=0, keepdims=True)
    out_ref[0, 2:3, :] = jnp.sum(onehot * conf, axis=0, keepdims=True)


def _finish_kernel(part_ref, out_ref):
    p = part_ref[...]
    num = jnp.sum(p[:, 0, :], axis=0, keepdims=True)
    sacc = jnp.sum(p[:, 1, :], axis=0, keepdims=True)
    sconf = jnp.sum(p[:, 2, :], axis=0, keepdims=True)
    safe_n = jnp.maximum(num, 1.0)
    acc_bin = sacc / safe_n
    conf_bin = sconf / safe_n
    has = num > 0.0
    ece = jnp.sum(jnp.where(has, jnp.abs(conf_bin - acc_bin) * num, 0.0))
    out_ref[0:1, :] = jnp.full_like(num, ece)
    out_ref[1:2, :] = jnp.where(has, acc_bin * num, 0.0)
    out_ref[2:3, :] = jnp.where(has, num, 0.0)


def kernel(probs, labels, mode):
    n, c = probs.shape
    r = ROWS_PER_BLOCK
    grid = n // r

    bb = jnp.linspace(0.0, 1.0, N_BINS + 1)
    lo = jnp.full((1, 128), 2.0, dtype=jnp.float32).at[0, :N_BINS].set(bb[:-1])
    hi = jnp.full((1, 128), -1.0, dtype=jnp.float32).at[0, :N_BINS].set(bb[1:])
    labels2 = labels.reshape(n, 1)

    partials = pl.pallas_call(
        _partials_kernel,
        grid=(grid,),
        in_specs=[
            pl.BlockSpec((1, 128), lambda i: (0, 0)),
            pl.BlockSpec((1, 128), lambda i: (0, 0)),
            pl.BlockSpec((r, c), lambda i: (i, 0)),
            pl.BlockSpec((r, 1), lambda i: (i, 0)),
        ],
        out_specs=pl.BlockSpec((1, 3, 128), lambda i: (i, 0, 0)),
        out_shape=jax.ShapeDtypeStruct((grid, 3, 128), jnp.float32),
        compiler_params=pltpu.CompilerParams(
            dimension_semantics=("parallel",),
        ),
    )(lo, hi, probs, labels2)

    out = pl.pallas_call(
        _finish_kernel,
        out_shape=jax.ShapeDtypeStruct((8, 128), jnp.float32),
    )(partials)

    ece = out[0, 0:1]
    correct = out[1, 0:N_BINS]
    num = out[2, 0:N_BINS]
    return (ece, correct, num)


# speedup vs baseline: 1.0634x; 1.0634x over previous
import jax
import jax.numpy as jnp
from jax.experimental import pallas as pl
from jax.experimental.pallas import tpu as pltpu

N_BINS = 15
ROWS_PER_BLOCK = 1000
N_STRIPES = 4


def _make_partials_kernel(n_stripes):
    def _partials_kernel(*refs):
        lo_ref, hi_ref = refs[0], refs[1]
        probs_refs = refs[2:2 + n_stripes]
        labels_refs = refs[2 + n_stripes:2 + 2 * n_stripes]
        out_ref = refs[-1]

        lo = lo_ref[...]
        hi = hi_ref[...]
        num_p = jnp.zeros((1, 128), jnp.float32)
        acc_p = jnp.zeros((1, 128), jnp.float32)
        conf_p = jnp.zeros((1, 128), jnp.float32)
        for p_ref, l_ref in zip(probs_refs, labels_refs):
            x = p_ref[...]
            conf = jnp.max(x, axis=1, keepdims=True)
            col = jax.lax.broadcasted_iota(jnp.int32, x.shape, 1)
            pred = jnp.min(jnp.where(x == conf, col, x.shape[1]), axis=1,
                           keepdims=True)
            acc = (pred == l_ref[...]).astype(jnp.float32)
            onehot = ((conf > lo) & (conf <= hi)).astype(jnp.float32)
            num_p = num_p + jnp.sum(onehot, axis=0, keepdims=True)
            acc_p = acc_p + jnp.sum(onehot * acc, axis=0, keepdims=True)
            conf_p = conf_p + jnp.sum(onehot * conf, axis=0, keepdims=True)

        out_ref[0, 0:1, :] = num_p
        out_ref[0, 1:2, :] = acc_p
        out_ref[0, 2:3, :] = conf_p

    return _partials_kernel


def _finish_kernel(part_ref, out_ref):
    p = part_ref[...]
    num = jnp.sum(p[:, 0, :], axis=0, keepdims=True)
    sacc = jnp.sum(p[:, 1, :], axis=0, keepdims=True)
    sconf = jnp.sum(p[:, 2, :], axis=0, keepdims=True)
    safe_n = jnp.maximum(num, 1.0)
    acc_bin = sacc / safe_n
    conf_bin = sconf / safe_n
    has = num > 0.0
    ece = jnp.sum(jnp.where(has, jnp.abs(conf_bin - acc_bin) * num, 0.0))
    out_ref[0:1, :] = jnp.full_like(num, ece)
    out_ref[1:2, :] = jnp.where(has, acc_bin * num, 0.0)
    out_ref[2:3, :] = jnp.where(has, num, 0.0)


def kernel(probs, labels, mode):
    n, c = probs.shape
    r = ROWS_PER_BLOCK
    k = N_STRIPES
    grid = n // (r * k)

    bb = jnp.linspace(0.0, 1.0, N_BINS + 1)
    lo = jnp.full((1, 128), 2.0, dtype=jnp.float32).at[0, :N_BINS].set(bb[:-1])
    hi = jnp.full((1, 128), -1.0, dtype=jnp.float32).at[0, :N_BINS].set(bb[1:])
    labels2 = labels.reshape(n, 1)

    probs_specs = [
        pl.BlockSpec((r, c), lambda i, s=s: (s * grid + i, 0)) for s in range(k)
    ]
    labels_specs = [
        pl.BlockSpec((r, 1), lambda i, s=s: (s * grid + i, 0)) for s in range(k)
    ]

    partials = pl.pallas_call(
        _make_partials_kernel(k),
        grid=(grid,),
        in_specs=[
            pl.BlockSpec((1, 128), lambda i: (0, 0)),
            pl.BlockSpec((1, 128), lambda i: (0, 0)),
            *probs_specs,
            *labels_specs,
        ],
        out_specs=pl.BlockSpec((1, 3, 128), lambda i: (i, 0, 0)),
        out_shape=jax.ShapeDtypeStruct((grid, 3, 128), jnp.float32),
        compiler_params=pltpu.CompilerParams(
            dimension_semantics=("arbitrary",),
        ),
    )(lo, hi, *([probs] * k), *([labels2] * k))

    out = pl.pallas_call(
        _finish_kernel,
        out_shape=jax.ShapeDtypeStruct((8, 128), jnp.float32),
    )(partials)

    ece = out[0, 0:1]
    correct = out[1, 0:N_BINS]
    num = out[2, 0:N_BINS]
    return (ece, correct, num)


# speedup vs baseline: 1.0726x; 1.0087x over previous
import jax
import jax.numpy as jnp
from jax.experimental import pallas as pl
from jax.experimental.pallas import tpu as pltpu

N_BINS = 15
ROWS_PER_BLOCK = 1000
NBUF = 8


def _ece_kernel(lo_ref, hi_ref, probs_ref, labels_ref, out_ref,
                pbuf, lbuf, psem, lsem):
    n, c = probs_ref.shape
    r = ROWS_PER_BLOCK
    nblk = n // r

    def start_copy(block, slot):
        pltpu.make_async_copy(
            probs_ref.at[pl.ds(block * r, r), :], pbuf.at[slot], psem.at[slot]
        ).start()
        pltpu.make_async_copy(
            labels_ref.at[pl.ds(block * r, r), :], lbuf.at[slot], lsem.at[slot]
        ).start()

    for b in range(NBUF):
        start_copy(b, b)

    lo = lo_ref[...]
    hi = hi_ref[...]

    def body(i, carry):
        num_p, acc_p, conf_p = carry
        slot = jax.lax.rem(i, NBUF)
        pltpu.make_async_copy(
            probs_ref.at[pl.ds(i * r, r), :], pbuf.at[slot], psem.at[slot]
        ).wait()
        pltpu.make_async_copy(
            labels_ref.at[pl.ds(i * r, r), :], lbuf.at[slot], lsem.at[slot]
        ).wait()

        x = pbuf[slot]
        lab = lbuf[slot]
        conf = jnp.max(x, axis=1, keepdims=True)
        col = jax.lax.broadcasted_iota(jnp.int32, x.shape, 1)
        pred = jnp.min(jnp.where(x == conf, col, c), axis=1, keepdims=True)
        acc = (pred == lab).astype(jnp.float32)
        onehot = ((conf > lo) & (conf <= hi)).astype(jnp.float32)

        @pl.when(i + NBUF < nblk)
        def _next():
            start_copy(i + NBUF, slot)

        return (num_p + jnp.sum(onehot, axis=0, keepdims=True),
                acc_p + jnp.sum(onehot * acc, axis=0, keepdims=True),
                conf_p + jnp.sum(onehot * conf, axis=0, keepdims=True))

    zeros = jnp.zeros((1, 128), jnp.float32)
    num, sacc, sconf = jax.lax.fori_loop(0, nblk, body, (zeros, zeros, zeros))

    safe_n = jnp.maximum(num, 1.0)
    acc_bin = sacc / safe_n
    conf_bin = sconf / safe_n
    has = num > 0.0
    ece = jnp.sum(jnp.where(has, jnp.abs(conf_bin - acc_bin) * num, 0.0))
    out_ref[0:1, :] = jnp.full_like(num, ece)
    out_ref[1:2, :] = jnp.where(has, acc_bin * num, 0.0)
    out_ref[2:3, :] = jnp.where(has, num, 0.0)


def kernel(probs, labels, mode):
    n, c = probs.shape
    r = ROWS_PER_BLOCK

    bb = jnp.linspace(0.0, 1.0, N_BINS + 1)
    lo = jnp.full((1, 128), 2.0, dtype=jnp.float32).at[0, :N_BINS].set(bb[:-1])
    hi = jnp.full((1, 128), -1.0, dtype=jnp.float32).at[0, :N_BINS].set(bb[1:])
    labels2 = labels.reshape(n, 1)

    out = pl.pallas_call(
        _ece_kernel,
        in_specs=[
            pl.BlockSpec(memory_space=pltpu.MemorySpace.VMEM),
            pl.BlockSpec(memory_space=pltpu.MemorySpace.VMEM),
            pl.BlockSpec(memory_space=pltpu.MemorySpace.HBM),
            pl.BlockSpec(memory_space=pltpu.MemorySpace.HBM),
        ],
        out_specs=pl.BlockSpec(memory_space=pltpu.MemorySpace.VMEM),
        out_shape=jax.ShapeDtypeStruct((8, 128), jnp.float32),
        scratch_shapes=[
            pltpu.VMEM((NBUF, r, c), jnp.float32),
            pltpu.VMEM((NBUF, r, 1), jnp.int32),
            pltpu.SemaphoreType.DMA((NBUF,)),
            pltpu.SemaphoreType.DMA((NBUF,)),
        ],
    )(lo, hi, probs, labels2)

    ece = out[0, 0:1]
    correct = out[1, 0:N_BINS]
    num = out[2, 0:N_BINS]
    return (ece, correct, num)


# speedup vs baseline: 1.0727x; 1.0001x over previous
import jax
import jax.numpy as jnp
from jax.experimental import pallas as pl
from jax.experimental.pallas import tpu as pltpu

N_BINS = 15
ROWS_PER_BLOCK = 5000
NBUF = 2


def _ece_kernel(lo_ref, hi_ref, probs_ref, labels_ref, out_ref,
                pbuf, lbuf, psem, lsem):
    n, c = probs_ref.shape
    r = ROWS_PER_BLOCK
    nblk = n // r

    def start_copy(block, slot):
        pltpu.make_async_copy(
            probs_ref.at[pl.ds(block * r, r), :], pbuf.at[slot], psem.at[slot]
        ).start()
        pltpu.make_async_copy(
            labels_ref.at[pl.ds(block * r, r), :], lbuf.at[slot], lsem.at[slot]
        ).start()

    for b in range(NBUF):
        start_copy(b, b)

    lo = lo_ref[...]
    hi = hi_ref[...]

    def body(i, carry):
        num_p, acc_p, conf_p = carry
        slot = jax.lax.rem(i, NBUF)
        pltpu.make_async_copy(
            probs_ref.at[pl.ds(i * r, r), :], pbuf.at[slot], psem.at[slot]
        ).wait()
        pltpu.make_async_copy(
            labels_ref.at[pl.ds(i * r, r), :], lbuf.at[slot], lsem.at[slot]
        ).wait()

        x = pbuf[slot]
        lab = lbuf[slot]
        conf = jnp.max(x, axis=1, keepdims=True)
        col = jax.lax.broadcasted_iota(jnp.int32, x.shape, 1)
        pred = jnp.min(jnp.where(x == conf, col, c), axis=1, keepdims=True)
        acc = (pred == lab).astype(jnp.float32)
        onehot = ((conf > lo) & (conf <= hi)).astype(jnp.float32)

        @pl.when(i + NBUF < nblk)
        def _next():
            start_copy(i + NBUF, slot)

        return (num_p + jnp.sum(onehot, axis=0, keepdims=True),
                acc_p + jnp.sum(onehot * acc, axis=0, keepdims=True),
                conf_p + jnp.sum(onehot * conf, axis=0, keepdims=True))

    zeros = jnp.zeros((1, 128), jnp.float32)
    num, sacc, sconf = jax.lax.fori_loop(0, nblk, body, (zeros, zeros, zeros))

    safe_n = jnp.maximum(num, 1.0)
    acc_bin = sacc / safe_n
    conf_bin = sconf / safe_n
    has = num > 0.0
    ece = jnp.sum(jnp.where(has, jnp.abs(conf_bin - acc_bin) * num, 0.0))
    out_ref[0:1, :] = jnp.full_like(num, ece)
    out_ref[1:2, :] = jnp.where(has, acc_bin * num, 0.0)
    out_ref[2:3, :] = jnp.where(has, num, 0.0)


def kernel(probs, labels, mode):
    n, c = probs.shape
    r = ROWS_PER_BLOCK

    bb = jnp.linspace(0.0, 1.0, N_BINS + 1)
    lo = jnp.full((1, 128), 2.0, dtype=jnp.float32).at[0, :N_BINS].set(bb[:-1])
    hi = jnp.full((1, 128), -1.0, dtype=jnp.float32).at[0, :N_BINS].set(bb[1:])
    labels2 = labels.reshape(n, 1)

    out = pl.pallas_call(
        _ece_kernel,
        in_specs=[
            pl.BlockSpec(memory_space=pltpu.MemorySpace.VMEM),
            pl.BlockSpec(memory_space=pltpu.MemorySpace.VMEM),
            pl.BlockSpec(memory_space=pltpu.MemorySpace.HBM),
            pl.BlockSpec(memory_space=pltpu.MemorySpace.HBM),
        ],
        out_specs=pl.BlockSpec(memory_space=pltpu.MemorySpace.VMEM),
        out_shape=jax.ShapeDtypeStruct((8, 128), jnp.float32),
        scratch_shapes=[
            pltpu.VMEM((NBUF, r, c), jnp.float32),
            pltpu.VMEM((NBUF, r, 1), jnp.int32),
            pltpu.SemaphoreType.DMA((NBUF,)),
            pltpu.SemaphoreType.DMA((NBUF,)),
        ],
    )(lo, hi, probs, labels2)

    ece = out[0, 0:1]
    correct = out[1, 0:N_BINS]
    num = out[2, 0:N_BINS]
    return (ece, correct, num)
